# baseline (speedup 1.0000x reference)
import functools

import jax
import jax.numpy as jnp
from jax import lax
from jax.experimental import pallas as pl
from jax.experimental.pallas import tpu as pltpu
from jax.experimental.pallas import tpu_sc as plsc

NC = 2
NS = 16
LANES = 16

D = 64
R_SC = 25600
TRASH = R_SC
C = 384
G = C // 128
ROWS_PER_SUB = R_SC // NS
EP_CHUNK = 320
LEAKY_SLOPE = 0.5


@functools.lru_cache(maxsize=None)
def _make_spmm(e_pad: int, r_pad: int, npass: int):
    e_w = e_pad // NS
    n_chunks = e_w // C

    mesh = plsc.VectorSubcoreMesh(core_axis_name="c", subcore_axis_name="s")

    @functools.partial(
        pl.kernel,
        out_type=jax.ShapeDtypeStruct((r_pad, D), jnp.float32),
        mesh=mesh,
        scratch_types=[
            pltpu.VMEM((3, G, 128), jnp.int32),
            pltpu.VMEM((C,), jnp.float32),
            pltpu.VMEM((G, 128), jnp.int32),
            pltpu.VMEM((C, D), jnp.float32),
            pltpu.VMEM_SHARED((R_SC + 8, D), jnp.float32),
            pltpu.SemaphoreType.DMA,
        ],
        compiler_params=pltpu.CompilerParams(
            use_tc_tiling_on_sc=False, needs_layout_passes=False),
    )
    def spmm(x_hbm, meta_hbm, zeros_hbm, out_hbm,
             meta_v, val_v, idx_v, rows_v, acc_sh, sem):
        c = lax.axis_index("c")
        s = lax.axis_index("s")

        for p in range(npass):
            lo = (p * NC + c) * R_SC

            pltpu.sync_copy(
                zeros_hbm, acc_sh.at[pl.ds(s * ROWS_PER_SUB, ROWS_PER_SUB)])
            plsc.subcore_barrier()

            def chunk_body(i, _):
                cid = s * n_chunks + i
                pltpu.sync_copy(meta_hbm.at[cid], meta_v)
                copies = [
                    pltpu.async_copy(
                        x_hbm.at[meta_v.at[1, g]],
                        rows_v.at[pl.ds(g * 128, 128)], sem)
                    for g in range(G)
                ]
                for g in range(G):
                    for t in range(128 // LANES):
                        sl = pl.ds(t * LANES, LANES)
                        esl = pl.ds(g * 128 + t * LANES, LANES)
                        loc = meta_v[0, g, sl] - lo
                        inr = (loc >= 0) & (loc < R_SC)
                        idx_v[g, sl] = jnp.where(inr, loc, TRASH)
                        vals = plsc.bitcast(meta_v[2, g, sl], jnp.float32)
                        val_v[esl] = jnp.where(inr, vals, 0.0)
                for cp in copies:
                    cp.wait()

                def scale_body(gi, _):
                    vvec = val_v[pl.ds(gi * LANES, LANES)]
                    for j in range(LANES):
                        r = gi * LANES + j
                        vv = jnp.broadcast_to(vvec[j], (LANES,))
                        for q in range(D // LANES):
                            sl = pl.ds(q * LANES, LANES)
                            rows_v[r, sl] = rows_v[r, sl] * vv
                    return 0
                lax.fori_loop(0, C // LANES, scale_body, 0)

                for g in range(G):
                    pltpu.sync_copy(
                        rows_v.at[pl.ds(g * 128, 128)],
                        acc_sh.at[idx_v.at[g]], add=True)
                return 0

            lax.fori_loop(0, n_chunks, chunk_body, 0)
            plsc.subcore_barrier()

            for z in range(ROWS_PER_SUB // EP_CHUNK):
                row0 = s * ROWS_PER_SUB + z * EP_CHUNK
                pltpu.sync_copy(acc_sh.at[pl.ds(row0, EP_CHUNK)],
                                rows_v.at[pl.ds(0, EP_CHUNK)])

                def ep_body(r, _):
                    for q in range(D // LANES):
                        sl = pl.ds(q * LANES, LANES)
                        xv = rows_v[r, sl]
                        rows_v[r, sl] = jnp.where(
                            xv >= 0.0, xv, xv * LEAKY_SLOPE)
                    return 0
                lax.fori_loop(0, EP_CHUNK, ep_body, 0, unroll=2)

                out0 = pl.multiple_of(lo + row0, 8)
                pltpu.sync_copy(rows_v.at[pl.ds(0, EP_CHUNK)],
                                out_hbm.at[pl.ds(out0, EP_CHUNK)])
            if p != npass - 1:
                plsc.subcore_barrier()

    return spmm


def _spmm_leaky(x, dst, src, val, r_out):
    e = dst.shape[0]
    step = NS * C
    e_pad = ((e + step - 1) // step) * step
    pad = e_pad - e
    dst_p = jnp.pad(dst.astype(jnp.int32), (0, pad)).reshape(-1, G, 128)
    src_p = jnp.pad(src.astype(jnp.int32), (0, pad)).reshape(-1, G, 128)
    val_p = lax.bitcast_convert_type(
        jnp.pad(val, (0, pad)), jnp.int32).reshape(-1, G, 128)
    meta = jnp.stack([dst_p, src_p, val_p], axis=1)
    npass = -(-r_out // (NC * R_SC))
    r_pad = npass * NC * R_SC
    zeros = jnp.zeros((ROWS_PER_SUB, D), jnp.float32)
    out = _make_spmm(e_pad, r_pad, npass)(x, meta, zeros)
    return out[:r_out]


def kernel(adj_indices, adj_values, social_indices, social_values,
           tag_indices, tag_values, keepRate, uEmbeds, iEmbeds, tEmbeds):
    del keepRate
    U, I, T = uEmbeds.shape[0], iEmbeds.shape[0], tEmbeds.shape[0]
    embeds = jnp.concatenate([uEmbeds, iEmbeds], axis=0)
    tagembeds = jnp.concatenate([iEmbeds, tEmbeds], axis=0)
    lats = [embeds]
    gnnLats, tagLats, socialLats = [], [tagembeds], []
    for _ in range(2):
        tem = _spmm_leaky(lats[-1], adj_indices[0], adj_indices[1],
                          adj_values, U + I)
        tagIn = jnp.concatenate([lats[-1][U:], tagLats[-1][I:]], axis=0)
        tagLat = _spmm_leaky(tagIn, tag_indices[0], tag_indices[1],
                             tag_values, I + T)
        socialULat = _spmm_leaky(lats[-1][:U], social_indices[0],
                                 social_indices[1], social_values, U)
        gnnLats.append(tem)
        tagLats.append(tagLat)
        socialLats.append(socialULat)
        lats.append(jnp.concatenate(
            [tem[:U] + socialULat, tem[U:] + tagLat[:I]], axis=0))
    out = lats[0] + lats[1] + lats[2]
    return (out, tuple(gnnLats), tuple(tagLats), tuple(socialLats))


# speedup vs baseline: 1.1470x; 1.1470x over previous
import functools

import jax
import jax.numpy as jnp
from jax import lax
from jax.experimental import pallas as pl
from jax.experimental.pallas import tpu as pltpu
from jax.experimental.pallas import tpu_sc as plsc

NC = 2
NS = 16
LANES = 16

D = 64
R_SC = 25600
TRASH = R_SC
C = 256
G = C // 128
ROWS_PER_SUB = R_SC // NS
EP_CHUNK = 200
LEAKY_SLOPE = 0.5


@functools.lru_cache(maxsize=None)
def _make_spmm(e_pad: int, r_pad: int, npass: int):
    e_w = e_pad // NS
    n_chunks = e_w // C

    mesh = plsc.VectorSubcoreMesh(core_axis_name="c", subcore_axis_name="s")

    @functools.partial(
        pl.kernel,
        out_type=jax.ShapeDtypeStruct((r_pad, D), jnp.float32),
        mesh=mesh,
        scratch_types=[
            pltpu.VMEM((3, G, 128), jnp.int32),
            pltpu.VMEM((C,), jnp.float32),
            pltpu.VMEM((G, 128), jnp.int32),
            pltpu.VMEM((C, D), jnp.float32),
            pltpu.VMEM_SHARED((R_SC + 8, D), jnp.float32),
            pltpu.SemaphoreType.DMA,
            pltpu.SemaphoreType.DMA,
        ],
        compiler_params=pltpu.CompilerParams(
            use_tc_tiling_on_sc=False, needs_layout_passes=False),
    )
    def spmm(x_hbm, meta_hbm, zeros_hbm, out_hbm,
             meta_v, val_v, idx_v, rows_v, acc_sh, sem, ssem):
        c = lax.axis_index("c")
        s = lax.axis_index("s")

        for p in range(npass):
            lo = (p * NC + c) * R_SC

            pltpu.sync_copy(
                zeros_hbm, acc_sh.at[pl.ds(s * ROWS_PER_SUB, ROWS_PER_SUB)])
            plsc.subcore_barrier()

            def chunk_body(i, _):
                cid = s * n_chunks + i
                pltpu.sync_copy(meta_hbm.at[cid], meta_v)
                copies = [
                    pltpu.async_copy(
                        x_hbm.at[meta_v.at[1, g]],
                        rows_v.at[pl.ds(g * 128, 128)], sem)
                    for g in range(G)
                ]
                for g in range(G):
                    for t in range(128 // LANES):
                        sl = pl.ds(t * LANES, LANES)
                        esl = pl.ds(g * 128 + t * LANES, LANES)
                        loc = meta_v[0, g, sl] - lo
                        inr = (loc >= 0) & (loc < R_SC)
                        idx_v[g, sl] = jnp.where(inr, loc, TRASH)
                        vals = plsc.bitcast(meta_v[2, g, sl], jnp.float32)
                        val_v[esl] = jnp.where(inr, vals, 0.0)
                def scale_body(gi, _):
                    vvec = val_v[pl.ds(gi * LANES, LANES)]
                    for j in range(LANES):
                        r = gi * LANES + j
                        vv = jnp.broadcast_to(vvec[j], (LANES,))
                        for q in range(D // LANES):
                            sl = pl.ds(q * LANES, LANES)
                            rows_v[r, sl] = rows_v[r, sl] * vv
                    return 0

                scats = []
                for g in range(G):
                    copies[g].wait()
                    lax.fori_loop(g * (128 // LANES), (g + 1) * (128 // LANES),
                                  scale_body, 0)
                    scats.append(pltpu.async_copy(
                        rows_v.at[pl.ds(g * 128, 128)],
                        acc_sh.at[idx_v.at[g]], ssem, add=True))
                for sc_ in scats:
                    sc_.wait()
                return 0

            lax.fori_loop(0, n_chunks, chunk_body, 0)
            plsc.subcore_barrier()

            for z in range(ROWS_PER_SUB // EP_CHUNK):
                row0 = s * ROWS_PER_SUB + z * EP_CHUNK
                pltpu.sync_copy(acc_sh.at[pl.ds(row0, EP_CHUNK)],
                                rows_v.at[pl.ds(0, EP_CHUNK)])

                def ep_body(r, _):
                    for q in range(D // LANES):
                        sl = pl.ds(q * LANES, LANES)
                        xv = rows_v[r, sl]
                        rows_v[r, sl] = jnp.where(
                            xv >= 0.0, xv, xv * LEAKY_SLOPE)
                    return 0
                lax.fori_loop(0, EP_CHUNK, ep_body, 0, unroll=2)

                out0 = pl.multiple_of(lo + row0, 8)
                pltpu.sync_copy(rows_v.at[pl.ds(0, EP_CHUNK)],
                                out_hbm.at[pl.ds(out0, EP_CHUNK)])
            if p != npass - 1:
                plsc.subcore_barrier()

    return spmm


def _spmm_leaky(x, dst, src, val, r_out):
    e = dst.shape[0]
    step = NS * C
    e_pad = ((e + step - 1) // step) * step
    pad = e_pad - e
    dst_p = jnp.pad(dst.astype(jnp.int32), (0, pad)).reshape(-1, G, 128)
    src_p = jnp.pad(src.astype(jnp.int32), (0, pad)).reshape(-1, G, 128)
    val_p = lax.bitcast_convert_type(
        jnp.pad(val, (0, pad)), jnp.int32).reshape(-1, G, 128)
    meta = jnp.stack([dst_p, src_p, val_p], axis=1)
    npass = -(-r_out // (NC * R_SC))
    r_pad = npass * NC * R_SC
    zeros = jnp.zeros((ROWS_PER_SUB, D), jnp.float32)
    out = _make_spmm(e_pad, r_pad, npass)(x, meta, zeros)
    return out[:r_out]


def kernel(adj_indices, adj_values, social_indices, social_values,
           tag_indices, tag_values, keepRate, uEmbeds, iEmbeds, tEmbeds):
    del keepRate
    U, I, T = uEmbeds.shape[0], iEmbeds.shape[0], tEmbeds.shape[0]
    embeds = jnp.concatenate([uEmbeds, iEmbeds], axis=0)
    tagembeds = jnp.concatenate([iEmbeds, tEmbeds], axis=0)
    lats = [embeds]
    gnnLats, tagLats, socialLats = [], [tagembeds], []
    for _ in range(2):
        tem = _spmm_leaky(lats[-1], adj_indices[0], adj_indices[1],
                          adj_values, U + I)
        tagIn = jnp.concatenate([lats[-1][U:], tagLats[-1][I:]], axis=0)
        tagLat = _spmm_leaky(tagIn, tag_indices[0], tag_indices[1],
                             tag_values, I + T)
        socialULat = _spmm_leaky(lats[-1][:U], social_indices[0],
                                 social_indices[1], social_values, U)
        gnnLats.append(tem)
        tagLats.append(tagLat)
        socialLats.append(socialULat)
        lats.append(jnp.concatenate(
            [tem[:U] + socialULat, tem[U:] + tagLat[:I]], axis=0))
    out = lats[0] + lats[1] + lats[2]
    return (out, tuple(gnnLats), tuple(tagLats), tuple(socialLats))


# speedup vs baseline: 1.7368x; 1.5142x over previous
import functools

import jax
import jax.numpy as jnp
from jax import lax
from jax.experimental import pallas as pl
from jax.experimental.pallas import tpu as pltpu
from jax.experimental.pallas import tpu_sc as plsc

NC = 2
NS = 16
LANES = 16

D = 64
R_SC = 25600
TRASH = R_SC
C = 256
G = C // 128
ROWS_PER_SUB = R_SC // NS
EP_CHUNK = 200
LEAKY_SLOPE = 0.5


@functools.lru_cache(maxsize=None)
def _make_spmm(e_pad: int, r_pad: int, npass: int):
    e_w = e_pad // NS
    n_chunks = e_w // C

    mesh = plsc.VectorSubcoreMesh(core_axis_name="c", subcore_axis_name="s")

    @functools.partial(
        pl.kernel,
        out_type=jax.ShapeDtypeStruct((r_pad, D), jnp.float32),
        mesh=mesh,
        scratch_types=[
            pltpu.VMEM((3, G, 128), jnp.int32),
            pltpu.VMEM((C,), jnp.float32),
            pltpu.VMEM((G, 128), jnp.int32),
            pltpu.VMEM((C, D), jnp.float32),
            pltpu.VMEM_SHARED((R_SC + 8, D), jnp.float32),
            pltpu.SemaphoreType.DMA,
            pltpu.SemaphoreType.DMA,
        ],
        compiler_params=pltpu.CompilerParams(
            use_tc_tiling_on_sc=False, needs_layout_passes=False),
    )
    def spmm(x_hbm, meta_hbm, zeros_hbm, out_hbm,
             meta_v, val_v, idx_v, rows_v, acc_sh, sem, ssem):
        c = lax.axis_index("c")
        s = lax.axis_index("s")

        for p in range(npass):
            lo = (p * NC + c) * R_SC

            pltpu.sync_copy(
                zeros_hbm, acc_sh.at[pl.ds(s * ROWS_PER_SUB, ROWS_PER_SUB)])
            plsc.subcore_barrier()

            def chunk_body(i, _):
                cid = s * n_chunks + i
                pltpu.sync_copy(meta_hbm.at[cid], meta_v)
                copies = [
                    pltpu.async_copy(
                        x_hbm.at[meta_v.at[1, g]],
                        rows_v.at[pl.ds(g * 128, 128)], sem)
                    for g in range(G)
                ]
                for g in range(G):
                    for t in range(128 // LANES):
                        sl = pl.ds(t * LANES, LANES)
                        esl = pl.ds(g * 128 + t * LANES, LANES)
                        loc = meta_v[0, g, sl] - lo
                        inr = (loc >= 0) & (loc < R_SC)
                        idx_v[g, sl] = jnp.where(inr, loc, TRASH)
                        vals = plsc.bitcast(meta_v[2, g, sl], jnp.float32)
                        val_v[esl] = jnp.where(inr, vals, 0.0)
                def scale_body(gi, _):
                    vvec = val_v[pl.ds(gi * LANES, LANES)]
                    for j in range(LANES):
                        r = gi * LANES + j
                        vv = jnp.broadcast_to(vvec[j], (LANES,))
                        for q in range(D // LANES):
                            sl = pl.ds(q * LANES, LANES)
                            rows_v[r, sl] = rows_v[r, sl] * vv
                    return 0

                scats = []
                for g in range(G):
                    copies[g].wait()
                    lax.fori_loop(g * (128 // LANES), (g + 1) * (128 // LANES),
                                  scale_body, 0, unroll=2)
                    scats.append(pltpu.async_copy(
                        rows_v.at[pl.ds(g * 128, 128)],
                        acc_sh.at[idx_v.at[g]], ssem, add=True))
                for sc_ in scats:
                    sc_.wait()
                return 0

            lax.fori_loop(0, n_chunks, chunk_body, 0)
            plsc.subcore_barrier()

            for z in range(ROWS_PER_SUB // EP_CHUNK):
                row0 = s * ROWS_PER_SUB + z * EP_CHUNK
                pltpu.sync_copy(acc_sh.at[pl.ds(row0, EP_CHUNK)],
                                rows_v.at[pl.ds(0, EP_CHUNK)])

                def ep_body(r, _):
                    for q in range(D // LANES):
                        sl = pl.ds(q * LANES, LANES)
                        xv = rows_v[r, sl]
                        rows_v[r, sl] = jnp.where(
                            xv >= 0.0, xv, xv * LEAKY_SLOPE)
                    return 0
                lax.fori_loop(0, EP_CHUNK, ep_body, 0, unroll=2)

                out0 = pl.multiple_of(lo + row0, 8)
                pltpu.sync_copy(rows_v.at[pl.ds(0, EP_CHUNK)],
                                out_hbm.at[pl.ds(out0, EP_CHUNK)])
            if p != npass - 1:
                plsc.subcore_barrier()

    return spmm


def _spmm_leaky(x, dst, src, val, r_out):
    e = dst.shape[0]
    step = NS * C
    e_pad = ((e + step - 1) // step) * step
    pad = e_pad - e
    dst_p = jnp.pad(dst.astype(jnp.int32), (0, pad)).reshape(-1, G, 128)
    src_p = jnp.pad(src.astype(jnp.int32), (0, pad)).reshape(-1, G, 128)
    val_p = lax.bitcast_convert_type(
        jnp.pad(val, (0, pad)), jnp.int32).reshape(-1, G, 128)
    meta = jnp.stack([dst_p, src_p, val_p], axis=1)
    npass = -(-r_out // (NC * R_SC))
    r_pad = npass * NC * R_SC
    zeros = jnp.zeros((ROWS_PER_SUB, D), jnp.float32)
    out = _make_spmm(e_pad, r_pad, npass)(x, meta, zeros)
    return out[:r_out]


def kernel(adj_indices, adj_values, social_indices, social_values,
           tag_indices, tag_values, keepRate, uEmbeds, iEmbeds, tEmbeds):
    del keepRate
    U, I, T = uEmbeds.shape[0], iEmbeds.shape[0], tEmbeds.shape[0]
    embeds = jnp.concatenate([uEmbeds, iEmbeds], axis=0)
    tagembeds = jnp.concatenate([iEmbeds, tEmbeds], axis=0)
    lats = [embeds]
    gnnLats, tagLats, socialLats = [], [tagembeds], []
    for _ in range(2):
        tem = _spmm_leaky(lats[-1], adj_indices[0], adj_indices[1],
                          adj_values, U + I)
        tagIn = jnp.concatenate([lats[-1][U:], tagLats[-1][I:]], axis=0)
        tagLat = _spmm_leaky(tagIn, tag_indices[0], tag_indices[1],
                             tag_values, I + T)
        socialULat = _spmm_leaky(lats[-1][:U], social_indices[0],
                                 social_indices[1], social_values, U)
        gnnLats.append(tem)
        tagLats.append(tagLat)
        socialLats.append(socialULat)
        lats.append(jnp.concatenate(
            [tem[:U] + socialULat, tem[U:] + tagLat[:I]], axis=0))
    out = lats[0] + lats[1] + lats[2]
    return (out, tuple(gnnLats), tuple(tagLats), tuple(socialLats))
